# Initial kernel scaffold; baseline (speedup 1.0000x reference)
import functools

import jax
import jax.numpy as jnp
from jax.experimental import pallas as pl
from jax.experimental.pallas import tpu as pltpu
from jax.experimental.pallas import tpu_sc as plsc

_GATHER_WIN = 128


def _sc_gather(id_emb, ids2d):
    n = ids2d.shape[1]
    d = id_emb.shape[1]
    mesh = plsc.VectorSubcoreMesh(core_axis_name="core", subcore_axis_name="subcore")

    @functools.partial(
        pl.kernel,
        out_type=jax.ShapeDtypeStruct((n, d), id_emb.dtype),
        mesh=mesh,
    )
    def gather_kernel(emb_hbm, ids_hbm, out_hbm):
        def body(i_vmem, o_vmem):
            pltpu.sync_copy(emb_hbm.at[i_vmem.at[0]], o_vmem)

        pltpu.emit_pipeline(
            body,
            grid=(n // _GATHER_WIN,),
            in_specs=[pl.BlockSpec((1, _GATHER_WIN), lambda i: (0, i))],
            out_specs=[pl.BlockSpec((_GATHER_WIN, d), lambda i: (i, 0))],
            core_axis_name=("core", "subcore"),
            dimension_semantics=(pltpu.PARALLEL,),
        )(ids_hbm, out_hbm)

    return gather_kernel(id_emb, ids2d)


_TC_BLK = 8192


def _tc_combine(tokens2d, pe, topo2d, Wt):
    n, d = tokens2d.shape
    f = topo2d.shape[1]

    def body(tok_ref, pe_ref, topo_ref, wt_ref, out_ref):
        out_ref[...] = tok_ref[...] + pe_ref[...] + jnp.dot(
            topo_ref[...], wt_ref[...], preferred_element_type=jnp.float32
        )

    return pl.pallas_call(
        body,
        grid=(n // _TC_BLK,),
        in_specs=[
            pl.BlockSpec((_TC_BLK, d), lambda i: (i, 0)),
            pl.BlockSpec((_TC_BLK, d), lambda i: (i, 0)),
            pl.BlockSpec((_TC_BLK, f), lambda i: (i, 0)),
            pl.BlockSpec((f, d), lambda i: (0, 0)),
        ],
        out_specs=pl.BlockSpec((_TC_BLK, d), lambda i: (i, 0)),
        out_shape=jax.ShapeDtypeStruct((n, d), jnp.float32),
    )(tokens2d, pe, topo2d, Wt)


def kernel(tokens, ids, topo_feats, id_emb, W_proj):
    b, s, d = tokens.shape
    n = b * s
    ids2d = ids.reshape(1, n).astype(jnp.int32)
    pe = _sc_gather(id_emb, ids2d)
    out = _tc_combine(
        tokens.reshape(n, d),
        pe,
        topo_feats.reshape(n, topo_feats.shape[-1]),
        W_proj.T,
    )
    return out.reshape(b, s, d)


# speedup vs baseline: 1.9300x; 1.9300x over previous
import functools

import jax
import jax.numpy as jnp
from jax.experimental import pallas as pl
from jax.experimental.pallas import tpu as pltpu
from jax.experimental.pallas import tpu_sc as plsc

_GATHER_WIN = 128


def _sc_gather(id_emb, ids2d):
    n = ids2d.shape[1]
    d = id_emb.shape[1]
    mesh = plsc.VectorSubcoreMesh(core_axis_name="core", subcore_axis_name="subcore")

    @functools.partial(
        pl.kernel,
        out_type=jax.ShapeDtypeStruct((n, d), id_emb.dtype),
        mesh=mesh,
        compiler_params=pltpu.CompilerParams(use_tc_tiling_on_sc=False),
    )
    def gather_kernel(emb_hbm, ids_hbm, out_hbm):
        def body(i_vmem, o_vmem):
            pltpu.sync_copy(emb_hbm.at[i_vmem.at[0]], o_vmem)

        pltpu.emit_pipeline(
            body,
            grid=(n // _GATHER_WIN,),
            in_specs=[pl.BlockSpec((1, _GATHER_WIN), lambda i: (0, i))],
            out_specs=[pl.BlockSpec((_GATHER_WIN, d), lambda i: (i, 0))],
            core_axis_name=("core", "subcore"),
            dimension_semantics=(pltpu.PARALLEL,),
        )(ids_hbm, out_hbm)

    return gather_kernel(id_emb, ids2d)


_TC_BLK = 8192


def _tc_combine(tokens2d, pe, topo2d, Wt):
    n, d = tokens2d.shape
    f = topo2d.shape[1]

    def body(tok_ref, pe_ref, topo_ref, wt_ref, out_ref):
        out_ref[...] = tok_ref[...] + pe_ref[...] + jnp.dot(
            topo_ref[...], wt_ref[...], preferred_element_type=jnp.float32
        )

    return pl.pallas_call(
        body,
        grid=(n // _TC_BLK,),
        in_specs=[
            pl.BlockSpec((_TC_BLK, d), lambda i: (i, 0)),
            pl.BlockSpec((_TC_BLK, d), lambda i: (i, 0)),
            pl.BlockSpec((_TC_BLK, f), lambda i: (i, 0)),
            pl.BlockSpec((f, d), lambda i: (0, 0)),
        ],
        out_specs=pl.BlockSpec((_TC_BLK, d), lambda i: (i, 0)),
        out_shape=jax.ShapeDtypeStruct((n, d), jnp.float32),
    )(tokens2d, pe, topo2d, Wt)


def kernel(tokens, ids, topo_feats, id_emb, W_proj):
    b, s, d = tokens.shape
    n = b * s
    ids2d = ids.reshape(1, n).astype(jnp.int32)
    pe = _sc_gather(id_emb, ids2d)
    out = _tc_combine(
        tokens.reshape(n, d),
        pe,
        topo_feats.reshape(n, topo_feats.shape[-1]),
        W_proj.T,
    )
    return out.reshape(b, s, d)


# speedup vs baseline: 2.0952x; 1.0856x over previous
import functools

import jax
import jax.numpy as jnp
from jax.experimental import pallas as pl
from jax.experimental.pallas import tpu as pltpu
from jax.experimental.pallas import tpu_sc as plsc

_GATHER_WIN = 128


def _sc_gather(table128, ids2d):
    n = ids2d.shape[1]
    d = table128.shape[1]
    mesh = plsc.VectorSubcoreMesh(core_axis_name="core", subcore_axis_name="subcore")

    @functools.partial(
        pl.kernel,
        out_type=jax.ShapeDtypeStruct((n, d), table128.dtype),
        mesh=mesh,
    )
    def gather_kernel(emb_hbm, ids_hbm, out_hbm):
        def body(i_vmem, o_vmem):
            pltpu.sync_copy(emb_hbm.at[i_vmem.at[0]], o_vmem)

        pltpu.emit_pipeline(
            body,
            grid=(n // _GATHER_WIN,),
            in_specs=[pl.BlockSpec((1, _GATHER_WIN), lambda i: (0, i))],
            out_specs=[pl.BlockSpec((_GATHER_WIN, d), lambda i: (i, 0))],
            core_axis_name=("core", "subcore"),
            dimension_semantics=(pltpu.PARALLEL,),
        )(ids_hbm, out_hbm)

    return gather_kernel(table128, ids2d)


_TC_BLK = 8192


def _tc_combine(tokens2d, pe_wide, topo2d, Wt):
    n, d = tokens2d.shape
    f = topo2d.shape[1]
    dw = pe_wide.shape[1]

    def body(tok_ref, pe_ref, topo_ref, wt_ref, out_ref):
        out_ref[...] = tok_ref[...] + pe_ref[:, :d] + jnp.dot(
            topo_ref[...], wt_ref[...], preferred_element_type=jnp.float32
        )

    return pl.pallas_call(
        body,
        grid=(n // _TC_BLK,),
        in_specs=[
            pl.BlockSpec((_TC_BLK, d), lambda i: (i, 0)),
            pl.BlockSpec((_TC_BLK, dw), lambda i: (i, 0)),
            pl.BlockSpec((_TC_BLK, f), lambda i: (i, 0)),
            pl.BlockSpec((f, d), lambda i: (0, 0)),
        ],
        out_specs=pl.BlockSpec((_TC_BLK, d), lambda i: (i, 0)),
        out_shape=jax.ShapeDtypeStruct((n, d), jnp.float32),
    )(tokens2d, pe_wide, topo2d, Wt)


def kernel(tokens, ids, topo_feats, id_emb, W_proj):
    b, s, d = tokens.shape
    n = b * s
    ids2d = ids.reshape(1, n).astype(jnp.int32)
    table128 = jnp.pad(id_emb, ((0, 0), (0, 128 - d)))
    pe_wide = _sc_gather(table128, ids2d)
    out = _tc_combine(
        tokens.reshape(n, d),
        pe_wide,
        topo_feats.reshape(n, topo_feats.shape[-1]),
        W_proj.T,
    )
    return out.reshape(b, s, d)


# speedup vs baseline: 2.4846x; 1.1858x over previous
import functools

import jax
import jax.numpy as jnp
from jax.experimental import pallas as pl
from jax.experimental.pallas import tpu as pltpu
from jax.experimental.pallas import tpu_sc as plsc

_GATHER_WIN = 128
_TABLE_BLK = 2000


def _eye(k):
    r = jax.lax.broadcasted_iota(jnp.int32, (k, k), 0)
    c = jax.lax.broadcasted_iota(jnp.int32, (k, k), 1)
    return (r == c).astype(jnp.float32)


def _sc_gather(table128, ids2d):
    n = ids2d.shape[1]
    dw = table128.shape[1]
    mesh = plsc.VectorSubcoreMesh(core_axis_name="core", subcore_axis_name="subcore")

    @functools.partial(
        pl.kernel,
        out_type=jax.ShapeDtypeStruct((n, dw), table128.dtype),
        mesh=mesh,
    )
    def gather_kernel(emb_hbm, ids_hbm, out_hbm):
        def body(i_vmem, o_vmem):
            pltpu.sync_copy(emb_hbm.at[i_vmem.at[0]], o_vmem)

        pltpu.emit_pipeline(
            body,
            grid=(n // _GATHER_WIN,),
            in_specs=[pl.BlockSpec((1, _GATHER_WIN), lambda i: (0, i))],
            out_specs=[pl.BlockSpec((_GATHER_WIN, dw), lambda i: (i, 0))],
            core_axis_name=("core", "subcore"),
            dimension_semantics=(pltpu.PARALLEL,),
        )(ids_hbm, out_hbm)

    return gather_kernel(table128, ids2d)


def _tc_combine(tokens_t, pe3, topo_t, W):
    s, d, b = tokens_t.shape
    f = topo_t.shape[1]
    dw = pe3.shape[2]

    def body(tok_ref, pe_ref, topo_ref, w_ref, out_ref):
        pe_t = jax.lax.dot_general(
            _eye(d), pe_ref[0, :, :d], (((1,), (1,)), ((), ())),
            preferred_element_type=jnp.float32,
        )
        proj = jax.lax.dot_general(
            w_ref[...], topo_ref[0], (((1,), (0,)), ((), ())),
            preferred_element_type=jnp.float32,
        )
        out_ref[0] = tok_ref[0] + pe_t + proj

    return pl.pallas_call(
        body,
        grid=(s,),
        in_specs=[
            pl.BlockSpec((1, d, b), lambda i: (i, 0, 0)),
            pl.BlockSpec((1, b, dw), lambda i: (i, 0, 0)),
            pl.BlockSpec((1, f, b), lambda i: (i, 0, 0)),
            pl.BlockSpec((d, f), lambda i: (0, 0)),
        ],
        out_specs=pl.BlockSpec((1, d, b), lambda i: (i, 0, 0)),
        out_shape=jax.ShapeDtypeStruct((s, d, b), jnp.float32),
    )(tokens_t, pe3, topo_t, W)


def kernel(tokens, ids, topo_feats, id_emb, W_proj):
    b, s, d = tokens.shape
    n = b * s
    tokens_t = jnp.transpose(tokens, (1, 2, 0))
    topo_t = jnp.transpose(topo_feats, (1, 2, 0))
    ids_sm = ids.T.reshape(1, n).astype(jnp.int32)
    table128 = jnp.pad(id_emb, ((0, 0), (0, 128 - d)))
    pe3 = _sc_gather(table128, ids_sm).reshape(s, b, 128)
    out_t = _tc_combine(tokens_t, pe3, topo_t, W_proj)
    return jnp.transpose(out_t, (2, 0, 1))
